# baseline (speedup 1.0000x reference)
import functools
import jax
import jax.numpy as jnp
from jax.experimental import pallas as pl
from jax.experimental.pallas import tpu as pltpu


def _tc_body(a_ref, p_ref, out_ref, *, batch, col_chunk):
    at = jax.lax.transpose(a_ref[...], (1, 0))
    pt = jax.lax.transpose(p_ref[...], (1, 0))
    a_nt = at * jax.lax.rsqrt(jnp.sum(at * at, axis=0, keepdims=True))
    p_nt = pt * jax.lax.rsqrt(jnp.sum(pt * pt, axis=0, keepdims=True))

    eye = (jax.lax.broadcasted_iota(jnp.int32, (col_chunk, col_chunk), 0) ==
           jax.lax.broadcasted_iota(jnp.int32, (col_chunk, col_chunk), 1))
    chunk_mins = []
    for c in range(batch // col_chunk):
        lo = c * col_chunk
        hi = lo + col_chunk
        s_c = jax.lax.dot_general(a_nt, p_nt[:, lo:hi],
                                  (((0,), (0,)), ((), ())),
                                  preferred_element_type=jnp.float32)
        parts = []
        if lo > 0:
            parts.append(jnp.min(s_c[:lo, :], axis=1, keepdims=True))
        mid = jnp.where(eye, jnp.inf, s_c[lo:hi, :])
        parts.append(jnp.min(mid, axis=1, keepdims=True))
        if hi < batch:
            parts.append(jnp.min(s_c[hi:, :], axis=1, keepdims=True))
        chunk_mins.append(jnp.concatenate(parts, axis=0))
    an = chunk_mins[0]
    for m in chunk_mins[1:]:
        an = jnp.minimum(an, m)
    ap = jnp.sum(a_nt * p_nt, axis=0, keepdims=True)
    ap_t = jax.lax.transpose(ap, (1, 0))
    loss = jnp.sum(jnp.maximum(1.0 + ap_t - an, 0.0)) * (1.0 / batch)
    out_ref[...] = jnp.full(out_ref.shape, loss, jnp.float32)


def kernel(anchor, positive):
    batch, dim = anchor.shape
    out = pl.pallas_call(
        functools.partial(_tc_body, batch=batch, col_chunk=2048),
        out_shape=jax.ShapeDtypeStruct((8, 128), jnp.float32),
        compiler_params=pltpu.CompilerParams(
            fuse_transposed_lhs_in_matmul=True),
    )(anchor, positive)
    return out[0, 0]


# speedup vs baseline: 1.5897x; 1.5897x over previous
import functools
import jax
import jax.numpy as jnp
from jax.experimental import pallas as pl
from jax.experimental.pallas import tpu as pltpu


def _tc_body(at_ref, pt_ref, out_ref, *, batch, col_chunk):
    at = at_ref[...]
    pt = pt_ref[...]
    a_nt = at * jax.lax.rsqrt(jnp.sum(at * at, axis=0, keepdims=True))
    p_nt = pt * jax.lax.rsqrt(jnp.sum(pt * pt, axis=0, keepdims=True))

    eye = (jax.lax.broadcasted_iota(jnp.int32, (col_chunk, col_chunk), 0) ==
           jax.lax.broadcasted_iota(jnp.int32, (col_chunk, col_chunk), 1))
    chunk_mins = []
    for c in range(batch // col_chunk):
        lo = c * col_chunk
        hi = lo + col_chunk
        s_c = jax.lax.dot_general(a_nt, p_nt[:, lo:hi],
                                  (((0,), (0,)), ((), ())),
                                  preferred_element_type=jnp.float32)
        parts = []
        if lo > 0:
            parts.append(jnp.min(s_c[:lo, :], axis=1, keepdims=True))
        mid = jnp.where(eye, jnp.inf, s_c[lo:hi, :])
        parts.append(jnp.min(mid, axis=1, keepdims=True))
        if hi < batch:
            parts.append(jnp.min(s_c[hi:, :], axis=1, keepdims=True))
        chunk_mins.append(jnp.concatenate(parts, axis=0))
    an = chunk_mins[0]
    for m in chunk_mins[1:]:
        an = jnp.minimum(an, m)
    ap = jnp.sum(a_nt * p_nt, axis=0, keepdims=True)
    ap_t = jax.lax.transpose(ap, (1, 0))
    loss = jnp.sum(jnp.maximum(1.0 + ap_t - an, 0.0)) * (1.0 / batch)
    out_ref[...] = jnp.full(out_ref.shape, loss, jnp.float32)


def kernel(anchor, positive):
    batch, dim = anchor.shape
    out = pl.pallas_call(
        functools.partial(_tc_body, batch=batch, col_chunk=2048),
        out_shape=jax.ShapeDtypeStruct((8, 128), jnp.float32),
        compiler_params=pltpu.CompilerParams(
            fuse_transposed_lhs_in_matmul=True),
    )(anchor.T, positive.T)
    return out[0, 0]


# speedup vs baseline: 1.5905x; 1.0005x over previous
import functools
import jax
import jax.numpy as jnp
from jax.experimental import pallas as pl
from jax.experimental.pallas import tpu as pltpu


def _tc_body(at_ref, pt_ref, out_ref, *, batch, col_chunk):
    at = at_ref[...]
    pt = pt_ref[...]
    a_nt = at * jax.lax.rsqrt(jnp.sum(at * at, axis=0, keepdims=True))
    p_nt = pt * jax.lax.rsqrt(jnp.sum(pt * pt, axis=0, keepdims=True))

    eye = (jax.lax.broadcasted_iota(jnp.int32, (col_chunk, col_chunk), 0) ==
           jax.lax.broadcasted_iota(jnp.int32, (col_chunk, col_chunk), 1))
    chunk_mins = []
    for c in range(batch // col_chunk):
        lo = c * col_chunk
        hi = lo + col_chunk
        s_c = jax.lax.dot_general(a_nt, p_nt[:, lo:hi],
                                  (((0,), (0,)), ((), ())),
                                  preferred_element_type=jnp.float32)
        parts = []
        if lo > 0:
            parts.append(jnp.min(s_c[:lo, :], axis=1, keepdims=True))
        mid = jnp.where(eye, jnp.inf, s_c[lo:hi, :])
        parts.append(jnp.min(mid, axis=1, keepdims=True))
        if hi < batch:
            parts.append(jnp.min(s_c[hi:, :], axis=1, keepdims=True))
        chunk_mins.append(jnp.concatenate(parts, axis=0))
    an = chunk_mins[0]
    for m in chunk_mins[1:]:
        an = jnp.minimum(an, m)
    ap = jnp.sum(a_nt * p_nt, axis=0, keepdims=True)
    ap_t = jax.lax.transpose(ap, (1, 0))
    loss = jnp.sum(jnp.maximum(1.0 + ap_t - an, 0.0)) * (1.0 / batch)
    out_ref[...] = jnp.full(out_ref.shape, loss, jnp.float32)


def kernel(anchor, positive):
    batch, dim = anchor.shape
    out = pl.pallas_call(
        functools.partial(_tc_body, batch=batch, col_chunk=2048),
        out_shape=jax.ShapeDtypeStruct((8, 128), jnp.float32),
    )(anchor.T, positive.T)
    return out[0, 0]


# speedup vs baseline: 1.5969x; 1.0040x over previous
import functools
import jax
import jax.numpy as jnp
from jax.experimental import pallas as pl
from jax.experimental.pallas import tpu as pltpu


def _tc_body(at_ref, pt_ref, out_ref, *, batch, col_chunk):
    at = at_ref[...]
    pt = pt_ref[...]
    a_nt = at * jax.lax.rsqrt(jnp.sum(at * at, axis=0, keepdims=True))
    p_nt = pt * jax.lax.rsqrt(jnp.sum(pt * pt, axis=0, keepdims=True))

    eye = (jax.lax.broadcasted_iota(jnp.int32, (col_chunk, col_chunk), 0) ==
           jax.lax.broadcasted_iota(jnp.int32, (col_chunk, col_chunk), 1))
    chunk_mins = []
    for c in range(batch // col_chunk):
        lo = c * col_chunk
        hi = lo + col_chunk
        s_c = jax.lax.dot_general(a_nt, p_nt[:, lo:hi],
                                  (((0,), (0,)), ((), ())),
                                  preferred_element_type=jnp.float32)
        parts = []
        if lo > 0:
            parts.append(jnp.min(s_c[:lo, :], axis=1, keepdims=True))
        mid = jnp.where(eye, jnp.inf, s_c[lo:hi, :])
        parts.append(jnp.min(mid, axis=1, keepdims=True))
        if hi < batch:
            parts.append(jnp.min(s_c[hi:, :], axis=1, keepdims=True))
        chunk_mins.append(jnp.concatenate(parts, axis=0))
    an = chunk_mins[0]
    for m in chunk_mins[1:]:
        an = jnp.minimum(an, m)
    ap = jnp.sum(a_nt * p_nt, axis=0, keepdims=True)
    ap_t = jax.lax.transpose(ap, (1, 0))
    loss = jnp.sum(jnp.maximum(1.0 + ap_t - an, 0.0)) * (1.0 / batch)
    out_ref[...] = jnp.full(out_ref.shape, loss, jnp.float32)


def kernel(anchor, positive):
    batch, dim = anchor.shape
    out = pl.pallas_call(
        functools.partial(_tc_body, batch=batch, col_chunk=2048),
        out_shape=jax.ShapeDtypeStruct((8, 128), jnp.float32),
        compiler_params=pltpu.CompilerParams(
            fuse_transposed_lhs_in_matmul=True),
    )(anchor.T, positive.T)
    return out[0, 0]


# speedup vs baseline: 1.6056x; 1.0054x over previous
import functools
import jax
import jax.numpy as jnp
from jax.experimental import pallas as pl
from jax.experimental.pallas import tpu as pltpu


def _tc_body(at_ref, pt_ref, out_ref, *, batch, col_chunk):
    at = at_ref[...]
    pt = pt_ref[...]
    a_nt = at * jax.lax.rsqrt(jnp.sum(at * at, axis=0, keepdims=True))
    p_nt = pt * jax.lax.rsqrt(jnp.sum(pt * pt, axis=0, keepdims=True))

    eye = (jax.lax.broadcasted_iota(jnp.int32, (col_chunk, col_chunk), 0) ==
           jax.lax.broadcasted_iota(jnp.int32, (col_chunk, col_chunk), 1))
    chunk_mins = []
    for c in range(batch // col_chunk):
        lo = c * col_chunk
        hi = lo + col_chunk
        s_c = jax.lax.dot_general(p_nt[:, lo:hi], a_nt,
                                  (((0,), (0,)), ((), ())),
                                  preferred_element_type=jnp.float32)
        parts = []
        if lo > 0:
            parts.append(jnp.min(s_c[:, :lo], axis=0, keepdims=True))
        mid = jnp.where(eye, jnp.inf, s_c[:, lo:hi])
        parts.append(jnp.min(mid, axis=0, keepdims=True))
        if hi < batch:
            parts.append(jnp.min(s_c[:, hi:], axis=0, keepdims=True))
        chunk_mins.append(jnp.concatenate(parts, axis=1))
    an = chunk_mins[0]
    for m in chunk_mins[1:]:
        an = jnp.minimum(an, m)
    ap = jnp.sum(a_nt * p_nt, axis=0, keepdims=True)
    loss = jnp.sum(jnp.maximum(1.0 + ap - an, 0.0)) * (1.0 / batch)
    out_ref[...] = jnp.full(out_ref.shape, loss, jnp.float32)


def kernel(anchor, positive):
    batch, dim = anchor.shape
    out = pl.pallas_call(
        functools.partial(_tc_body, batch=batch, col_chunk=2048),
        out_shape=jax.ShapeDtypeStruct((8, 128), jnp.float32),
        compiler_params=pltpu.CompilerParams(
            fuse_transposed_lhs_in_matmul=True),
    )(anchor.T, positive.T)
    return out[0, 0]
